# baseline (speedup 1.0000x reference)
import jax
import jax.numpy as jnp
from jax import lax
from jax.experimental import pallas as pl
from jax.experimental.pallas import tpu as pltpu
from jax.experimental.pallas import tpu_sc as plsc

_NUM_CORES = 2
_NUM_SUBCORES = 16
_NUM_WORKERS = _NUM_CORES * _NUM_SUBCORES
_FILL_BLK = 3200


def _worker_id():
    return lax.axis_index("s") * _NUM_CORES + lax.axis_index("c")


def _sc_gather_body(per_w, scores_hbm, ids_hbm, staged_hbm,
                    idx_v, vals_v, isem, gsem):
    base = _worker_id() * per_w
    pltpu.async_copy(ids_hbm.at[pl.ds(base, per_w)], idx_v.at[0], isem).wait()
    pltpu.async_copy(scores_hbm.at[idx_v.at[0]], vals_v, gsem).wait()
    pltpu.async_copy(vals_v, staged_hbm.at[pl.ds(base, per_w)], gsem).wait()


def _sc_scatter_body(per_w, ids_hbm, staged_hbm, out_hbm,
                     idx_v, vals_v, isem, gsem):
    base = _worker_id() * per_w
    ids_cp = pltpu.async_copy(ids_hbm.at[pl.ds(base, per_w)], idx_v.at[0], isem)
    vals_cp = pltpu.async_copy(
        staged_hbm.at[pl.ds(base, per_w)], vals_v, gsem)
    ids_cp.wait()
    vals_cp.wait()
    pltpu.async_copy(vals_v, out_hbm.at[idx_v.at[0]], gsem).wait()


def _fill_body(out_ref):
    out_ref[...] = jnp.full(out_ref.shape, -jnp.inf, dtype=jnp.float32)


def kernel(input_ids, scores, allowed_token_ids):
    del input_ids
    batch, vocab = scores.shape
    n_ids = allowed_token_ids.shape[0]
    per_w = n_ids // _NUM_WORKERS
    ids = allowed_token_ids.astype(jnp.int32)
    scores_t = scores.T

    mesh = plsc.VectorSubcoreMesh(core_axis_name="c", subcore_axis_name="s")
    sc_params = pltpu.CompilerParams(needs_layout_passes=False)

    def gather_body(*args):
        _sc_gather_body(per_w, *args)

    staged = pl.kernel(
        gather_body,
        out_type=jax.ShapeDtypeStruct((n_ids, batch), jnp.float32),
        mesh=mesh,
        scratch_types=[
            pltpu.VMEM((1, per_w), jnp.int32),
            pltpu.VMEM((per_w, batch), jnp.float32),
            pltpu.SemaphoreType.DMA,
            pltpu.SemaphoreType.DMA,
        ],
        compiler_params=sc_params,
        name="sc_gather_allowed_rows",
    )(scores_t, ids)

    filled_t = pl.pallas_call(
        _fill_body,
        grid=(-(-vocab // _FILL_BLK),),
        out_specs=pl.BlockSpec((_FILL_BLK, batch), lambda i: (i, 0)),
        out_shape=jax.ShapeDtypeStruct((vocab, batch), jnp.float32),
        name="tc_neg_inf_fill",
    )()

    def run(refs):
        ids_ref, staged_ref, out_ref = refs

        @pl.core_map(
            mesh,
            compiler_params=sc_params,
            scratch_shapes=[
                pltpu.VMEM((1, per_w), jnp.int32),
                pltpu.VMEM((per_w, batch), jnp.float32),
                pltpu.SemaphoreType.DMA,
                pltpu.SemaphoreType.DMA,
            ],
            name="sc_scatter_allowed_rows",
        )
        def _(*scratch):
            _sc_scatter_body(per_w, ids_ref, staged_ref, out_ref, *scratch)

    _, _, out_t = pl.run_state(run)((ids, staged, filled_t))
    return out_t.T


# speedup vs baseline: 1.1220x; 1.1220x over previous
import jax
import jax.numpy as jnp
from jax import lax
from jax.experimental import pallas as pl
from jax.experimental.pallas import tpu as pltpu
from jax.experimental.pallas import tpu_sc as plsc

_NUM_CORES = 2
_NUM_SUBCORES = 16
_NUM_WORKERS = _NUM_CORES * _NUM_SUBCORES
_FILL_BLK = 8192


def _worker_id():
    return lax.axis_index("s") * _NUM_CORES + lax.axis_index("c")


def _sc_gather_body(per_w, scores_hbm, ids_hbm, staged_hbm,
                    idx_v, vals_v, isem, gsem):
    base = _worker_id() * per_w
    pltpu.async_copy(ids_hbm.at[pl.ds(base, per_w)], idx_v.at[0], isem).wait()
    pltpu.async_copy(scores_hbm.at[idx_v.at[0]], vals_v, gsem).wait()
    pltpu.async_copy(vals_v, staged_hbm.at[pl.ds(base, per_w)], gsem).wait()


def _sc_scatter_body(per_w, ids_hbm, staged_hbm, out_hbm,
                     idx_v, vals_v, isem, gsem):
    base = _worker_id() * per_w
    ids_cp = pltpu.async_copy(ids_hbm.at[pl.ds(base, per_w)], idx_v.at[0], isem)
    vals_cp = pltpu.async_copy(
        staged_hbm.at[pl.ds(base, per_w)], vals_v, gsem)
    ids_cp.wait()
    vals_cp.wait()
    pltpu.async_copy(vals_v, out_hbm.at[idx_v.at[0]], gsem).wait()


def _fill_body(out_ref):
    out_ref[...] = jnp.full(out_ref.shape, -jnp.inf, dtype=jnp.float32)


def kernel(input_ids, scores, allowed_token_ids):
    del input_ids
    batch, vocab = scores.shape
    n_ids = allowed_token_ids.shape[0]
    per_w = n_ids // _NUM_WORKERS
    ids = allowed_token_ids.astype(jnp.int32)
    scores_t = scores.T

    mesh = plsc.VectorSubcoreMesh(core_axis_name="c", subcore_axis_name="s")
    sc_params = pltpu.CompilerParams(needs_layout_passes=False)

    def gather_body(*args):
        _sc_gather_body(per_w, *args)

    staged = pl.kernel(
        gather_body,
        out_type=jax.ShapeDtypeStruct((n_ids, batch), jnp.float32),
        mesh=mesh,
        scratch_types=[
            pltpu.VMEM((1, per_w), jnp.int32),
            pltpu.VMEM((per_w, batch), jnp.float32),
            pltpu.SemaphoreType.DMA,
            pltpu.SemaphoreType.DMA,
        ],
        compiler_params=sc_params,
        name="sc_gather_allowed_rows",
    )(scores_t, ids)

    filled_t = pl.pallas_call(
        _fill_body,
        grid=(-(-vocab // _FILL_BLK),),
        out_specs=pl.BlockSpec((_FILL_BLK, batch), lambda i: (i, 0)),
        out_shape=jax.ShapeDtypeStruct((vocab, batch), jnp.float32),
        name="tc_neg_inf_fill",
    )()

    def run(refs):
        ids_ref, staged_ref, out_ref = refs

        @pl.core_map(
            mesh,
            compiler_params=sc_params,
            scratch_shapes=[
                pltpu.VMEM((1, per_w), jnp.int32),
                pltpu.VMEM((per_w, batch), jnp.float32),
                pltpu.SemaphoreType.DMA,
                pltpu.SemaphoreType.DMA,
            ],
            name="sc_scatter_allowed_rows",
        )
        def _(*scratch):
            _sc_scatter_body(per_w, ids_ref, staged_ref, out_ref, *scratch)

    _, _, out_t = pl.run_state(run)((ids, staged, filled_t))
    return out_t.T
